# baseline (speedup 1.0000x reference)
import jax
import jax.numpy as jnp
from jax.experimental import pallas as pl
from jax.experimental.pallas import tpu as pltpu


def _gcn_block_kernel(x_ref, w_ref, adj_a_ref, adj_b_ref, out_ref, support_ref):
    tm = adj_a_ref.shape[0]

    @pl.when(pl.program_id(0) == 0)
    def _():
        support_ref[...] = jnp.dot(
            x_ref[...], w_ref[...], preferred_element_type=jnp.float32
        )

    out_ref[:tm, :] = jnp.dot(
        adj_a_ref[...], support_ref[...], preferred_element_type=jnp.float32
    )
    out_ref[tm:, :] = jnp.dot(
        adj_b_ref[...], support_ref[...], preferred_element_type=jnp.float32
    )


def _pick_row_tile(n):
    for tm in (200, 1000, 40, 8):
        if n % (2 * tm) == 0:
            return tm
    return None


@jax.jit
def kernel(x, adj, W):
    n, d_in = x.shape
    d_out = W.shape[1]
    tm = _pick_row_tile(n)
    grid = (n // (2 * tm),)

    return pl.pallas_call(
        _gcn_block_kernel,
        grid=grid,
        in_specs=[
            pl.BlockSpec((n, d_in), lambda i: (0, 0)),
            pl.BlockSpec((d_in, d_out), lambda i: (0, 0)),
            pl.BlockSpec((tm, n), lambda i: (2 * i, 0)),
            pl.BlockSpec((tm, n), lambda i: (2 * i + 1, 0)),
        ],
        out_specs=pl.BlockSpec((2 * tm, d_out), lambda i: (i, 0)),
        out_shape=jax.ShapeDtypeStruct((n, d_out), jnp.float32),
        scratch_shapes=[pltpu.VMEM((n, d_out), jnp.float32)],
        compiler_params=pltpu.CompilerParams(
            dimension_semantics=("arbitrary",),
        ),
    )(x, W, adj, adj)


# speedup vs baseline: 1.0183x; 1.0183x over previous
import jax
import jax.numpy as jnp
from jax.experimental import pallas as pl
from jax.experimental.pallas import tpu as pltpu


def _gcn_block_kernel(x_ref, w_ref, adj_ref, out_ref, support_ref):
    @pl.when(pl.program_id(0) == 0)
    def _():
        support_ref[...] = jnp.dot(
            x_ref[...], w_ref[...], preferred_element_type=jnp.float32
        )

    out_ref[...] = jnp.dot(
        adj_ref[...], support_ref[...], preferred_element_type=jnp.float32
    )


def _pick_row_tile(n):
    for tm in (400, 512, 256, 200, 128, 100, 80, 64, 40, 16, 8):
        if n % tm == 0:
            return tm
    return n


@jax.jit
def kernel(x, adj, W):
    n, d_in = x.shape
    d_out = W.shape[1]
    tm = _pick_row_tile(n)
    grid = (n // tm,)

    return pl.pallas_call(
        _gcn_block_kernel,
        grid=grid,
        in_specs=[
            pl.BlockSpec((n, d_in), lambda i: (0, 0)),
            pl.BlockSpec((d_in, d_out), lambda i: (0, 0)),
            pl.BlockSpec((tm, n), lambda i: (i, 0)),
        ],
        out_specs=pl.BlockSpec((tm, d_out), lambda i: (i, 0)),
        out_shape=jax.ShapeDtypeStruct((n, d_out), jnp.float32),
        scratch_shapes=[pltpu.VMEM((n, d_out), jnp.float32)],
        compiler_params=pltpu.CompilerParams(
            dimension_semantics=("arbitrary",),
        ),
    )(x, W, adj)
